# baseline (speedup 1.0000x reference)
import functools

import jax
import jax.numpy as jnp
from jax import lax
from jax.experimental import pallas as pl
from jax.experimental.pallas import tpu as pltpu
from jax.experimental.pallas import tpu_sc as plsc

N = 10000
D = 128
H = 64
HW = H // 2
E2 = 320000

NC = 2
NS = 16
L = 16
NW = NC * NS
EW = E2 // NW
B = 400
NCHUNK = EW // B


def _proj_body(x_ref, w_ref, m_ref):
    m_ref[...] = jnp.dot(x_ref[...], w_ref[...],
                         preferred_element_type=jnp.float32)


def _encode_body(adj_ref, m_ref, w2_ref, z_ref, y_ref):
    a = adj_ref[...].astype(jnp.bfloat16)
    m = m_ref[...].astype(jnp.bfloat16)
    z = jnp.dot(a, m, preferred_element_type=jnp.float32)
    z_ref[...] = z.astype(jnp.bfloat16)
    y_ref[...] = (z * w2_ref[...]).astype(jnp.bfloat16)


def _decode_body(z_hbm, y_hbm, ii_hbm, jj_hbm, out_hbm,
                 ii_all, jj_all, ri0, rj0, ri1, rj1, out0, out1,
                 sem_r0, sem_r1, sem_o0, sem_o1):
    wid = lax.axis_index("s") * NC + lax.axis_index("c")
    wbase = wid * EW

    pltpu.sync_copy(ii_hbm.at[pl.ds(wbase, EW)], ii_all)
    pltpu.sync_copy(jj_hbm.at[pl.ds(wbase, EW)], jj_all)

    def start_gather(c, ri, rj, sem):
        off = c * B
        pltpu.async_copy(z_hbm.at[ii_all.at[pl.ds(off, B)]], ri, sem)
        pltpu.async_copy(y_hbm.at[jj_all.at[pl.ds(off, B)]], rj, sem)

    def wait_gather(c, ri, rj, sem):
        off = c * B
        pltpu.make_async_copy(z_hbm.at[ii_all.at[pl.ds(off, B)]], ri, sem).wait()
        pltpu.make_async_copy(y_hbm.at[jj_all.at[pl.ds(off, B)]], rj, sem).wait()

    dnums = lax.GatherDimensionNumbers(
        offset_dims=(), collapsed_slice_dims=(0,), start_index_map=(0,))

    def lane_shuffle(v, idx):
        return lax.gather(v, idx[:, None], dnums, slice_sizes=(1,),
                          mode=lax.GatherScatterMode.PROMISE_IN_BOUNDS)

    def compute(c, ri, rj, out_v, sem_o):
        lanes = lax.iota(jnp.int32, L)

        himask = jnp.full((L,), jnp.int32(-65536))

        @pl.loop(0, B // L)
        def _group(g):
            e0 = g * L
            acc = jnp.zeros((L,), jnp.float32)
            for l in range(L):
                e = e0 + l
                p = None
                for k in range(HW // L):
                    zi = ri[e, pl.ds(k * L, L)]
                    yj = rj[e, pl.ds(k * L, L)]
                    zlo = plsc.bitcast(zi << 16, jnp.float32)
                    zhi = plsc.bitcast(zi & himask, jnp.float32)
                    ylo = plsc.bitcast(yj << 16, jnp.float32)
                    yhi = plsc.bitcast(yj & himask, jnp.float32)
                    q = zlo * ylo + zhi * yhi
                    p = q if p is None else p + q
                for s in (1, 2, 4, 8):
                    p = p + lane_shuffle(p, lanes ^ s)
                acc = jnp.where(lanes == l, p, acc)
            out_v[pl.ds(e0, L)] = 1.0 / (1.0 + jnp.exp(-acc))

        pltpu.async_copy(
            out_v.at[pl.ds(0, B)], out_hbm.at[pl.ds(wbase + c * B, B)], sem_o)

    def wait_out(c, out_v, sem_o):
        pltpu.make_async_copy(
            out_v.at[pl.ds(0, B)], out_hbm.at[pl.ds(wbase + c * B, B)],
            sem_o).wait()

    start_gather(0, ri0, rj0, sem_r0)

    @pl.loop(0, (NCHUNK - 1) // 2)
    def _pair(it):
        c0 = 2 * it
        c1 = 2 * it + 1
        start_gather(c1, ri1, rj1, sem_r1)
        wait_gather(c0, ri0, rj0, sem_r0)
        pl.when(it > 0)(lambda: wait_out(c0 - 2, out0, sem_o0))
        compute(c0, ri0, rj0, out0, sem_o0)
        start_gather(c1 + 1, ri0, rj0, sem_r0)
        wait_gather(c1, ri1, rj1, sem_r1)
        pl.when(it > 0)(lambda: wait_out(c1 - 2, out1, sem_o1))
        compute(c1, ri1, rj1, out1, sem_o1)

    wait_gather(NCHUNK - 1, ri0, rj0, sem_r0)
    wait_out(NCHUNK - 3, out0, sem_o0)
    compute(NCHUNK - 1, ri0, rj0, out0, sem_o0)
    wait_out(NCHUNK - 2, out1, sem_o1)
    wait_out(NCHUNK - 1, out0, sem_o0)


def kernel(X, train_edges, train_false_edges, adj, weight, weight_two):
    m = pl.pallas_call(
        _proj_body,
        grid=(5,),
        in_specs=[
            pl.BlockSpec((2000, D), lambda i: (i, 0)),
            pl.BlockSpec((D, H), lambda i: (0, 0)),
        ],
        out_specs=pl.BlockSpec((2000, H), lambda i: (i, 0)),
        out_shape=jax.ShapeDtypeStruct((N, H), jnp.float32),
    )(X, weight)

    w2row = weight_two.reshape(1, H)
    BM = 400
    z, y = pl.pallas_call(
        _encode_body,
        grid=(N // BM,),
        in_specs=[
            pl.BlockSpec((BM, N), lambda i: (i, 0)),
            pl.BlockSpec((N, H), lambda i: (0, 0)),
            pl.BlockSpec((1, H), lambda i: (0, 0)),
        ],
        out_specs=[
            pl.BlockSpec((BM, H), lambda i: (i, 0)),
            pl.BlockSpec((BM, H), lambda i: (i, 0)),
        ],
        out_shape=[
            jax.ShapeDtypeStruct((N, H), jnp.bfloat16),
            jax.ShapeDtypeStruct((N, H), jnp.bfloat16),
        ],
    )(adj, m, w2row)

    z32 = lax.bitcast_convert_type(z.reshape(N, HW, 2), jnp.int32)
    y32 = lax.bitcast_convert_type(y.reshape(N, HW, 2), jnp.int32)

    ii = jnp.concatenate([train_edges[:, 0], train_false_edges[:, 0]])
    jj = jnp.concatenate([train_edges[:, 1], train_false_edges[:, 1]])

    mesh = plsc.VectorSubcoreMesh(core_axis_name="c", subcore_axis_name="s")
    decode = functools.partial(
        pl.kernel,
        out_type=jax.ShapeDtypeStruct((E2,), jnp.float32),
        mesh=mesh,
        compiler_params=pltpu.CompilerParams(
            needs_layout_passes=False, use_tc_tiling_on_sc=False),
        scratch_types=[
            pltpu.VMEM((EW,), jnp.int32),
            pltpu.VMEM((EW,), jnp.int32),
            pltpu.VMEM((B, HW), jnp.int32),
            pltpu.VMEM((B, HW), jnp.int32),
            pltpu.VMEM((B, HW), jnp.int32),
            pltpu.VMEM((B, HW), jnp.int32),
            pltpu.VMEM((B + L,), jnp.float32),
            pltpu.VMEM((B + L,), jnp.float32),
            pltpu.SemaphoreType.DMA,
            pltpu.SemaphoreType.DMA,
            pltpu.SemaphoreType.DMA,
            pltpu.SemaphoreType.DMA,
        ],
    )(_decode_body)

    out = decode(z32, y32, ii, jj)
    return out.reshape(E2, 1)


# speedup vs baseline: 1.1607x; 1.1607x over previous
import functools

import jax
import jax.numpy as jnp
from jax import lax
from jax.experimental import pallas as pl
from jax.experimental.pallas import tpu as pltpu
from jax.experimental.pallas import tpu_sc as plsc

N = 10000
D = 128
H = 64
HW = H // 2
E2 = 320000

NC = 2
NS = 16
L = 16
NW = NC * NS
EW = E2 // NW
B = 400
NCHUNK = EW // B


def _pack_pairs(v):
    lo = lax.convert_element_type(
        lax.bitcast_convert_type(v[:, :HW], jnp.uint16), jnp.int32)
    hi = lax.convert_element_type(
        lax.bitcast_convert_type(v[:, HW:], jnp.uint16), jnp.int32)
    return lo | (hi << 16)


def _encode_body(x_ref, w_ref, w2_ref, adj_ref, z_ref, y_ref, m_ref):
    @pl.when(pl.program_id(0) == 0)
    def _():
        m_ref[...] = jnp.dot(
            x_ref[...], w_ref[...],
            preferred_element_type=jnp.float32).astype(jnp.bfloat16)

    a = adj_ref[...].astype(jnp.bfloat16)
    z = jnp.dot(a, m_ref[...], preferred_element_type=jnp.float32)
    z_ref[...] = _pack_pairs(z.astype(jnp.bfloat16))
    y_ref[...] = _pack_pairs((z * w2_ref[...]).astype(jnp.bfloat16))


def _decode_body(z_hbm, y_hbm, ii_hbm, jj_hbm, out_hbm,
                 ii_all, jj_all, ri0, rj0, ri1, rj1, out0, out1,
                 sem_r0, sem_r1, sem_o0, sem_o1):
    wid = lax.axis_index("s") * NC + lax.axis_index("c")
    wbase = wid * EW

    pltpu.sync_copy(ii_hbm.at[pl.ds(wbase, EW)], ii_all)
    pltpu.sync_copy(jj_hbm.at[pl.ds(wbase, EW)], jj_all)

    def start_gather(c, ri, rj, sem):
        off = c * B
        pltpu.async_copy(z_hbm.at[ii_all.at[pl.ds(off, B)]], ri, sem)
        pltpu.async_copy(y_hbm.at[jj_all.at[pl.ds(off, B)]], rj, sem)

    def wait_gather(c, ri, rj, sem):
        off = c * B
        pltpu.make_async_copy(z_hbm.at[ii_all.at[pl.ds(off, B)]], ri, sem).wait()
        pltpu.make_async_copy(y_hbm.at[jj_all.at[pl.ds(off, B)]], rj, sem).wait()

    dnums = lax.GatherDimensionNumbers(
        offset_dims=(), collapsed_slice_dims=(0,), start_index_map=(0,))

    def lane_shuffle(v, idx):
        return lax.gather(v, idx[:, None], dnums, slice_sizes=(1,),
                          mode=lax.GatherScatterMode.PROMISE_IN_BOUNDS)

    def compute(c, ri, rj, out_v, sem_o):
        lanes = lax.iota(jnp.int32, L)

        himask = jnp.full((L,), jnp.int32(-65536))

        @pl.loop(0, B // L)
        def _group(g):
            e0 = g * L
            acc = jnp.zeros((L,), jnp.float32)
            for l in range(L):
                e = e0 + l
                p = None
                for k in range(HW // L):
                    zi = ri[e, pl.ds(k * L, L)]
                    yj = rj[e, pl.ds(k * L, L)]
                    zlo = plsc.bitcast(zi << 16, jnp.float32)
                    zhi = plsc.bitcast(zi & himask, jnp.float32)
                    ylo = plsc.bitcast(yj << 16, jnp.float32)
                    yhi = plsc.bitcast(yj & himask, jnp.float32)
                    q = zlo * ylo + zhi * yhi
                    p = q if p is None else p + q
                for s in (1, 2, 4, 8):
                    p = p + lane_shuffle(p, lanes ^ s)
                acc = jnp.where(lanes == l, p, acc)
            out_v[pl.ds(e0, L)] = 1.0 / (1.0 + jnp.exp(-acc))

        pltpu.async_copy(
            out_v.at[pl.ds(0, B)], out_hbm.at[pl.ds(wbase + c * B, B)], sem_o)

    def wait_out(c, out_v, sem_o):
        pltpu.make_async_copy(
            out_v.at[pl.ds(0, B)], out_hbm.at[pl.ds(wbase + c * B, B)],
            sem_o).wait()

    start_gather(0, ri0, rj0, sem_r0)

    @pl.loop(0, (NCHUNK - 1) // 2)
    def _pair(it):
        c0 = 2 * it
        c1 = 2 * it + 1
        start_gather(c1, ri1, rj1, sem_r1)
        wait_gather(c0, ri0, rj0, sem_r0)
        pl.when(it > 0)(lambda: wait_out(c0 - 2, out0, sem_o0))
        compute(c0, ri0, rj0, out0, sem_o0)
        start_gather(c1 + 1, ri0, rj0, sem_r0)
        wait_gather(c1, ri1, rj1, sem_r1)
        pl.when(it > 0)(lambda: wait_out(c1 - 2, out1, sem_o1))
        compute(c1, ri1, rj1, out1, sem_o1)

    wait_gather(NCHUNK - 1, ri0, rj0, sem_r0)
    wait_out(NCHUNK - 3, out0, sem_o0)
    compute(NCHUNK - 1, ri0, rj0, out0, sem_o0)
    wait_out(NCHUNK - 2, out1, sem_o1)
    wait_out(NCHUNK - 1, out0, sem_o0)


def kernel(X, train_edges, train_false_edges, adj, weight, weight_two):
    w2row = weight_two.reshape(1, H)
    BM = 400
    z32, y32 = pl.pallas_call(
        _encode_body,
        grid=(N // BM,),
        in_specs=[
            pl.BlockSpec((N, D), lambda i: (0, 0)),
            pl.BlockSpec((D, H), lambda i: (0, 0)),
            pl.BlockSpec((1, H), lambda i: (0, 0)),
            pl.BlockSpec((BM, N), lambda i: (i, 0)),
        ],
        out_specs=[
            pl.BlockSpec((BM, HW), lambda i: (i, 0)),
            pl.BlockSpec((BM, HW), lambda i: (i, 0)),
        ],
        out_shape=[
            jax.ShapeDtypeStruct((N, HW), jnp.int32),
            jax.ShapeDtypeStruct((N, HW), jnp.int32),
        ],
        scratch_shapes=[pltpu.VMEM((N, H), jnp.bfloat16)],
    )(X, weight, w2row, adj)

    ii = jnp.concatenate([train_edges[:, 0], train_false_edges[:, 0]])
    jj = jnp.concatenate([train_edges[:, 1], train_false_edges[:, 1]])

    mesh = plsc.VectorSubcoreMesh(core_axis_name="c", subcore_axis_name="s")
    decode = functools.partial(
        pl.kernel,
        out_type=jax.ShapeDtypeStruct((E2,), jnp.float32),
        mesh=mesh,
        compiler_params=pltpu.CompilerParams(
            needs_layout_passes=False, use_tc_tiling_on_sc=False),
        scratch_types=[
            pltpu.VMEM((EW,), jnp.int32),
            pltpu.VMEM((EW,), jnp.int32),
            pltpu.VMEM((B, HW), jnp.int32),
            pltpu.VMEM((B, HW), jnp.int32),
            pltpu.VMEM((B, HW), jnp.int32),
            pltpu.VMEM((B, HW), jnp.int32),
            pltpu.VMEM((B + L,), jnp.float32),
            pltpu.VMEM((B + L,), jnp.float32),
            pltpu.SemaphoreType.DMA,
            pltpu.SemaphoreType.DMA,
            pltpu.SemaphoreType.DMA,
            pltpu.SemaphoreType.DMA,
        ],
    )(_decode_body)

    out = decode(z32, y32, ii, jj)
    return out.reshape(E2, 1)


# speedup vs baseline: 1.2287x; 1.0586x over previous
import functools

import jax
import jax.numpy as jnp
from jax import lax
from jax.experimental import pallas as pl
from jax.experimental.pallas import tpu as pltpu
from jax.experimental.pallas import tpu_sc as plsc

N = 10000
D = 128
H = 64
HW = H // 2
E2 = 320000

NC = 2
NS = 16
L = 16
NW = NC * NS
EW = E2 // NW
B = 400
NCHUNK = EW // B


def _pack_pairs(v):
    lo = lax.convert_element_type(
        lax.bitcast_convert_type(v[:, :HW], jnp.uint16), jnp.int32)
    hi = lax.convert_element_type(
        lax.bitcast_convert_type(v[:, HW:], jnp.uint16), jnp.int32)
    return lo | (hi << 16)


def _encode_body(x_ref, w_ref, w2_ref, adj_ref, z_ref, y_ref, m_ref):
    @pl.when(pl.program_id(0) == 0)
    def _():
        m_ref[...] = jnp.dot(
            x_ref[...], w_ref[...],
            preferred_element_type=jnp.float32).astype(jnp.bfloat16)

    a = adj_ref[...].astype(jnp.bfloat16)
    z = jnp.dot(a, m_ref[...], preferred_element_type=jnp.float32)
    z_ref[...] = _pack_pairs(z.astype(jnp.bfloat16))
    y_ref[...] = _pack_pairs((z * w2_ref[...]).astype(jnp.bfloat16))


def _decode_body(z_hbm, y_hbm, ii_hbm, jj_hbm, out_hbm,
                 ii_all, jj_all, ri0, rj0, ri1, rj1, out0, out1,
                 sem_r0, sem_r1, sem_o0, sem_o1):
    wid = lax.axis_index("s") * NC + lax.axis_index("c")
    wbase = wid * EW

    pltpu.sync_copy(ii_hbm.at[pl.ds(wbase, EW)], ii_all)
    pltpu.sync_copy(jj_hbm.at[pl.ds(wbase, EW)], jj_all)

    def start_gather(c, ri, rj, sem):
        off = c * B
        pltpu.async_copy(z_hbm.at[ii_all.at[pl.ds(off, B)]], ri, sem)
        pltpu.async_copy(y_hbm.at[jj_all.at[pl.ds(off, B)]], rj, sem)

    def wait_gather(c, ri, rj, sem):
        off = c * B
        pltpu.make_async_copy(z_hbm.at[ii_all.at[pl.ds(off, B)]], ri, sem).wait()
        pltpu.make_async_copy(y_hbm.at[jj_all.at[pl.ds(off, B)]], rj, sem).wait()

    dnums = lax.GatherDimensionNumbers(
        offset_dims=(), collapsed_slice_dims=(0,), start_index_map=(0,))

    def lane_shuffle(v, idx):
        return lax.gather(v, idx[:, None], dnums, slice_sizes=(1,),
                          mode=lax.GatherScatterMode.PROMISE_IN_BOUNDS)

    def compute(c, ri, rj, out_v, sem_o):
        lanes = lax.iota(jnp.int32, L)

        smasks = {s: (lanes & s) != 0 for s in (1, 2, 4, 8)}

        def combine(a, b, s):
            m = smasks[s]
            t = jnp.where(m, b, a)
            u = jnp.where(m, a, b)
            return t + lane_shuffle(u, lanes ^ s)

        @pl.loop(0, B // L)
        def _group(g):
            e0 = g * L
            ps = []
            for l in range(L):
                e = e0 + l
                p = None
                for k in range(HW // L):
                    zi = ri[e, pl.ds(k * L, L)]
                    yj = rj[e, pl.ds(k * L, L)]
                    zlo = plsc.bitcast(zi << 16, jnp.float32)
                    zhi = plsc.bitcast(zi, jnp.float32)
                    ylo = plsc.bitcast(yj << 16, jnp.float32)
                    yhi = plsc.bitcast(yj, jnp.float32)
                    q = zlo * ylo + zhi * yhi
                    p = q if p is None else p + q
                ps.append(p)
            s = 1
            while len(ps) > 1:
                ps = [combine(ps[i], ps[i + 1], s)
                      for i in range(0, len(ps), 2)]
                s *= 2
            out_v[pl.ds(e0, L)] = 1.0 / (1.0 + jnp.exp(-ps[0]))

        pltpu.async_copy(
            out_v.at[pl.ds(0, B)], out_hbm.at[pl.ds(wbase + c * B, B)], sem_o)

    def wait_out(c, out_v, sem_o):
        pltpu.make_async_copy(
            out_v.at[pl.ds(0, B)], out_hbm.at[pl.ds(wbase + c * B, B)],
            sem_o).wait()

    start_gather(0, ri0, rj0, sem_r0)

    @pl.loop(0, (NCHUNK - 1) // 2)
    def _pair(it):
        c0 = 2 * it
        c1 = 2 * it + 1
        start_gather(c1, ri1, rj1, sem_r1)
        wait_gather(c0, ri0, rj0, sem_r0)
        pl.when(it > 0)(lambda: wait_out(c0 - 2, out0, sem_o0))
        compute(c0, ri0, rj0, out0, sem_o0)
        start_gather(c1 + 1, ri0, rj0, sem_r0)
        wait_gather(c1, ri1, rj1, sem_r1)
        pl.when(it > 0)(lambda: wait_out(c1 - 2, out1, sem_o1))
        compute(c1, ri1, rj1, out1, sem_o1)

    wait_gather(NCHUNK - 1, ri0, rj0, sem_r0)
    wait_out(NCHUNK - 3, out0, sem_o0)
    compute(NCHUNK - 1, ri0, rj0, out0, sem_o0)
    wait_out(NCHUNK - 2, out1, sem_o1)
    wait_out(NCHUNK - 1, out0, sem_o0)


def kernel(X, train_edges, train_false_edges, adj, weight, weight_two):
    w2row = weight_two.reshape(1, H)
    BM = 400
    z32, y32 = pl.pallas_call(
        _encode_body,
        grid=(N // BM,),
        in_specs=[
            pl.BlockSpec((N, D), lambda i: (0, 0)),
            pl.BlockSpec((D, H), lambda i: (0, 0)),
            pl.BlockSpec((1, H), lambda i: (0, 0)),
            pl.BlockSpec((BM, N), lambda i: (i, 0)),
        ],
        out_specs=[
            pl.BlockSpec((BM, HW), lambda i: (i, 0)),
            pl.BlockSpec((BM, HW), lambda i: (i, 0)),
        ],
        out_shape=[
            jax.ShapeDtypeStruct((N, HW), jnp.int32),
            jax.ShapeDtypeStruct((N, HW), jnp.int32),
        ],
        scratch_shapes=[pltpu.VMEM((N, H), jnp.bfloat16)],
    )(X, weight, w2row, adj)

    ii = jnp.concatenate([train_edges[:, 0], train_false_edges[:, 0]])
    jj = jnp.concatenate([train_edges[:, 1], train_false_edges[:, 1]])

    mesh = plsc.VectorSubcoreMesh(core_axis_name="c", subcore_axis_name="s")
    decode = functools.partial(
        pl.kernel,
        out_type=jax.ShapeDtypeStruct((E2,), jnp.float32),
        mesh=mesh,
        compiler_params=pltpu.CompilerParams(
            needs_layout_passes=False, use_tc_tiling_on_sc=False),
        scratch_types=[
            pltpu.VMEM((EW,), jnp.int32),
            pltpu.VMEM((EW,), jnp.int32),
            pltpu.VMEM((B, HW), jnp.int32),
            pltpu.VMEM((B, HW), jnp.int32),
            pltpu.VMEM((B, HW), jnp.int32),
            pltpu.VMEM((B, HW), jnp.int32),
            pltpu.VMEM((B + L,), jnp.float32),
            pltpu.VMEM((B + L,), jnp.float32),
            pltpu.SemaphoreType.DMA,
            pltpu.SemaphoreType.DMA,
            pltpu.SemaphoreType.DMA,
            pltpu.SemaphoreType.DMA,
        ],
    )(_decode_body)

    out = decode(z32, y32, ii, jj)
    return out.reshape(E2, 1)
